# baseline (speedup 1.0000x reference)
import functools

import jax
import jax.numpy as jnp
from jax import lax
from jax.experimental import pallas as pl
from jax.experimental.pallas import tpu as pltpu
from jax.experimental.pallas import tpu_sc as plsc
from jax.scipy.special import gammaln

_N_STUDY = 500.0
_BLK = 8192
_BLK2 = 32768
_NBINS = 9
_SC_ROWS = 40960
_SC_NC = 2
_SC_NS = 16
_SC_CH = 64


def _matvec_kernel(consts_ref, W_ref, X_ref, z_ref, part_ref):
    alpha = consts_ref[0]
    w = W_ref[...]
    x = X_ref[...]
    z = jax.lax.dot_general(
        w, x, (((1,), (1,)), ((), ())),
        preferred_element_type=jnp.float32)
    mu = jnp.exp(z)
    a = mu * mu
    num = a * _N_STUDY
    den = a * (_N_STUDY * _N_STUDY)
    e = (alpha * num) / den
    z_ref[...] = z.reshape(1, 1, _BLK)
    lane = jax.lax.broadcasted_iota(jnp.int32, (1, 128), 1)
    acc = jnp.full((1, 128), 0.0, jnp.float32)
    for h in range(_BLK // 8192):
        sh = jnp.sum(e[:, h * 8192:(h + 1) * 8192])
        acc = jnp.where(lane == h, sh, acc)
    part_ref[...] = acc.reshape(1, 1, 128)


def _sc_matvec(x_hbm, w_hbm, p_hbm, xbuf0, xbuf1, pbuf, wbuf, sem0, sem1):
    nw = _SC_NC * _SC_NS
    rows_per_tile = _SC_ROWS // nw
    nch = rows_per_tile // _SC_CH
    npair = nch // 2
    wid = lax.axis_index("s") * _SC_NC + lax.axis_index("c")
    base = (x_hbm.shape[0] - _SC_ROWS) + wid * rows_per_tile

    pltpu.sync_copy(w_hbm, wbuf)
    wv = [wbuf[pl.ds(16 * k, 16)] for k in range(16)]

    def compute_chunk(xb, c):
        for r in range(_SC_CH):
            t = [xb[r, pl.ds(16 * k, 16)] * wv[k] for k in range(16)]
            while len(t) > 1:
                t = [t[2 * i] + t[2 * i + 1] for i in range(len(t) // 2)]
            pbuf[r, pl.ds(0, 16)] = t[0]
        pltpu.sync_copy(
            pbuf, p_hbm.at[pl.ds(wid * rows_per_tile + c * _SC_CH, _SC_CH)])

    pltpu.async_copy(x_hbm.at[pl.ds(base, _SC_CH)], xbuf0, sem0)

    def pair_body(g, _):
        c0 = 2 * g
        pltpu.make_async_copy(
            x_hbm.at[pl.ds(base, _SC_CH)], xbuf0, sem0).wait()
        pltpu.async_copy(
            x_hbm.at[pl.ds(base + (c0 + 1) * _SC_CH, _SC_CH)], xbuf1, sem1)
        compute_chunk(xbuf0, c0)
        pltpu.make_async_copy(
            x_hbm.at[pl.ds(base, _SC_CH)], xbuf1, sem1).wait()

        @pl.when(g < npair - 1)
        def _():
            pltpu.async_copy(
                x_hbm.at[pl.ds(base + (c0 + 2) * _SC_CH, _SC_CH)], xbuf0, sem0)

        compute_chunk(xbuf1, c0 + 1)
        return ()

    lax.fori_loop(0, npair, pair_body, ())


def _scred_kernel(consts_ref, p_ref, z8_ref, out_ref):
    alpha = consts_ref[0]
    p = p_ref[...]
    jj = jax.lax.broadcasted_iota(jnp.int32, (128, 128), 0)
    cc = jax.lax.broadcasted_iota(jnp.int32, (128, 128), 1)
    sel = jnp.where(jj // 16 == cc, 1.0, 0.0).astype(jnp.float32)
    z = jax.lax.dot_general(
        p, sel, (((1,), (0,)), ((), ())),
        preferred_element_type=jnp.float32)
    mu = jnp.exp(z)
    a = mu * mu
    num = a * _N_STUDY
    den = a * (_N_STUDY * _N_STUDY)
    e = (alpha * num) / den
    lane = jax.lax.broadcasted_iota(jnp.int32, z.shape, 1)
    em = jnp.where(lane < 8, e, 0.0)
    z8_ref[...] = z
    ng = _SC_ROWS // 8192
    l2 = jax.lax.broadcasted_iota(jnp.int32, (1, 128), 1)
    acc = jnp.full((1, 128), 0.0, jnp.float32)
    for g in range(ng):
        sg = jnp.sum(em[g * 1024:(g + 1) * 1024, :])
        acc = jnp.where(l2 == g, sg, acc)
    out_ref[...] = acc


def _nb_kernel(consts_ref, ibits_ref, z_ref, y_ref, out_ref):
    v = consts_ref[0]
    rmin_bits = ibits_ref[0]
    z = z_ref[...]
    mu = jnp.exp(z)
    a = mu * mu
    num = a * _N_STUDY
    den = a * (_N_STUDY * _N_STUDY)
    rr = (v * den) / num
    idx = jax.lax.bitcast_convert_type(rr, jnp.int32) - rmin_bits
    idx = jnp.clip(idx, 0, _NBINS - 1)

    p = num / (v * (mu * _N_STUDY) + num)
    yi = y_ref[...]
    yf = yi.astype(jnp.float32)
    nb = rr * jnp.log(1.0 - p) + yf * jnp.log(p)
    partial_nb = jnp.sum(nb)

    lane = jax.lax.broadcasted_iota(jnp.int32, (1, 128), 1)
    acc = jnp.full((1, 128), 0.0, jnp.float32)
    for j in range(_NBINS):
        mj = idx == j
        for k in (1, 2, 3):
            cnt = jnp.sum(jnp.where(mj & (yi == k), 1.0, 0.0))
            acc = jnp.where(lane == (j * 3 + k - 1), cnt, acc)
    acc = jnp.where(lane == _NBINS * 3, partial_nb, acc)
    out_ref[...] = acc.reshape(1, 1, 128)


def kernel(X, y, W, theta):
    n = jnp.float32(_N_STUDY)
    alpha = 100.0 * n * jax.nn.sigmoid(theta[0]) + 1e-8
    consts1 = jnp.stack([alpha, alpha]).astype(jnp.float32)

    nrows = X.shape[0]
    tc_rows = nrows - _SC_ROWS
    nblk = tc_rows // _BLK

    z3, part1 = pl.pallas_call(
        _matvec_kernel,
        grid=(nblk,),
        in_specs=[
            pl.BlockSpec(memory_space=pltpu.SMEM),
            pl.BlockSpec((1, 256), lambda i: (0, 0)),
            pl.BlockSpec((_BLK, 256), lambda i: (i, 0)),
        ],
        out_specs=[
            pl.BlockSpec((1, 1, _BLK), lambda i: (i, 0, 0)),
            pl.BlockSpec((1, 1, 128), lambda i: (i, 0, 0)),
        ],
        out_shape=[
            jax.ShapeDtypeStruct((nblk, 1, _BLK), jnp.float32),
            jax.ShapeDtypeStruct((nblk, 1, 128), jnp.float32),
        ],
        compiler_params=pltpu.CompilerParams(
            dimension_semantics=("parallel",)),
    )(consts1, W, X)

    sc_kernel = functools.partial(
        pl.kernel,
        out_type=jax.ShapeDtypeStruct((_SC_ROWS, 16), jnp.float32),
        mesh=plsc.VectorSubcoreMesh(core_axis_name="c", subcore_axis_name="s"),
        scratch_types=[
            pltpu.VMEM((_SC_CH, 256), jnp.float32),
            pltpu.VMEM((_SC_CH, 256), jnp.float32),
            pltpu.VMEM((_SC_CH, 16), jnp.float32),
            pltpu.VMEM((256,), jnp.float32),
            pltpu.SemaphoreType.DMA,
            pltpu.SemaphoreType.DMA,
        ],
    )(_sc_matvec)
    z_part = sc_kernel(X, W.reshape(256))

    z8, part_sc = pl.pallas_call(
        _scred_kernel,
        grid=(1,),
        in_specs=[
            pl.BlockSpec(memory_space=pltpu.SMEM),
            pl.BlockSpec((_SC_ROWS // 8, 128), lambda i: (0, 0)),
        ],
        out_specs=[
            pl.BlockSpec((_SC_ROWS // 8, 128), lambda i: (0, 0)),
            pl.BlockSpec((1, 128), lambda i: (0, 0)),
        ],
        out_shape=[
            jax.ShapeDtypeStruct((_SC_ROWS // 8, 128), jnp.float32),
            jax.ShapeDtypeStruct((1, 128), jnp.float32),
        ],
    )(consts1, z_part.reshape(_SC_ROWS // 8, 128))
    z_sc = z8[:, :8].reshape(_SC_ROWS)

    parts = jnp.concatenate(
        [part1[:, 0, : _BLK // 8192].reshape(tc_rows // 8192),
         part_sc[0, : _SC_ROWS // 8192]])
    est_alpha = jnp.sum(parts) / jnp.float32(nrows)
    v = 1.0 / est_alpha
    r = v * n
    rc_bits = jax.lax.bitcast_convert_type(r, jnp.int32)
    rmin_bits = rc_bits - (_NBINS // 2)
    cand = jax.lax.bitcast_convert_type(
        rmin_bits + jnp.arange(_NBINS, dtype=jnp.int32), jnp.float32)
    ks = jnp.arange(1, 4, dtype=jnp.float32)
    T = (gammaln(cand[:, None] + ks[None, :])
         - gammaln(ks + 1.0)[None, :] - gammaln(cand)[:, None])

    consts2 = jnp.stack([v, r]).astype(jnp.float32)
    ibits = rmin_bits.reshape(1).astype(jnp.int32)

    nblk2 = nrows // _BLK2
    sub2 = _BLK2 // 128
    z_flat = jnp.concatenate([z3.reshape(tc_rows), z_sc])
    z2 = z_flat.reshape(nrows // 128, 128)
    y2 = y.reshape(nrows // 128, 128)

    partials = pl.pallas_call(
        _nb_kernel,
        grid=(nblk2,),
        in_specs=[
            pl.BlockSpec(memory_space=pltpu.SMEM),
            pl.BlockSpec(memory_space=pltpu.SMEM),
            pl.BlockSpec((sub2, 128), lambda i: (i, 0)),
            pl.BlockSpec((sub2, 128), lambda i: (i, 0)),
        ],
        out_specs=pl.BlockSpec((1, 1, 128), lambda i: (i, 0, 0)),
        out_shape=jax.ShapeDtypeStruct((nblk2, 1, 128), jnp.float32),
        compiler_params=pltpu.CompilerParams(
            dimension_semantics=("parallel",)),
    )(consts2, ibits, z2, y2)

    lanes = jnp.sum(partials[:, 0, :], axis=0)
    counts = lanes[: _NBINS * 3].reshape(_NBINS, 3)
    s3 = jnp.sum(counts * T)
    l = s3 + lanes[_NBINS * 3]
    return -l


# speedup vs baseline: 1.0399x; 1.0399x over previous
import functools

import jax
import jax.numpy as jnp
from jax import lax
from jax.experimental import pallas as pl
from jax.experimental.pallas import tpu as pltpu
from jax.experimental.pallas import tpu_sc as plsc
from jax.scipy.special import gammaln

_N_STUDY = 500.0
_BLK = 8192
_BLK2 = 32768
_NBINS = 9
_SC_ROWS = 32768
_SC_NC = 2
_SC_NS = 16
_SC_CH = 64


def _matvec_kernel(consts_ref, W_ref, X_ref, z_ref, part_ref):
    alpha = consts_ref[0]
    w = W_ref[...]
    x = X_ref[...]
    z = jax.lax.dot_general(
        w, x, (((1,), (1,)), ((), ())),
        preferred_element_type=jnp.float32)
    mu = jnp.exp(z)
    a = mu * mu
    num = a * _N_STUDY
    den = a * (_N_STUDY * _N_STUDY)
    e = (alpha * num) / den
    z_ref[...] = z.reshape(1, 1, _BLK)
    lane = jax.lax.broadcasted_iota(jnp.int32, (1, 128), 1)
    part_ref[...] = jnp.where(lane == 0, jnp.sum(e), 0.0).reshape(1, 1, 128)


def _sc_matvec(x_hbm, w_hbm, p_hbm, xbuf0, xbuf1, pbuf, wbuf, sem0, sem1):
    nw = _SC_NC * _SC_NS
    rows_per_tile = _SC_ROWS // nw
    nch = rows_per_tile // _SC_CH
    npair = nch // 2
    wid = lax.axis_index("s") * _SC_NC + lax.axis_index("c")
    base = (x_hbm.shape[0] - _SC_ROWS) + wid * rows_per_tile

    pltpu.sync_copy(w_hbm, wbuf)
    wv = [wbuf[pl.ds(16 * k, 16)] for k in range(16)]

    def compute_chunk(xb, c):
        for r in range(_SC_CH):
            t = [xb[r, pl.ds(16 * k, 16)] * wv[k] for k in range(16)]
            while len(t) > 1:
                t = [t[2 * i] + t[2 * i + 1] for i in range(len(t) // 2)]
            pbuf[r, pl.ds(0, 16)] = t[0]
        pltpu.sync_copy(
            pbuf, p_hbm.at[pl.ds(wid * rows_per_tile + c * _SC_CH, _SC_CH)])

    pltpu.async_copy(x_hbm.at[pl.ds(base, _SC_CH)], xbuf0, sem0)

    def pair_body(g, _):
        c0 = 2 * g
        pltpu.make_async_copy(
            x_hbm.at[pl.ds(base, _SC_CH)], xbuf0, sem0).wait()
        pltpu.async_copy(
            x_hbm.at[pl.ds(base + (c0 + 1) * _SC_CH, _SC_CH)], xbuf1, sem1)
        compute_chunk(xbuf0, c0)
        pltpu.make_async_copy(
            x_hbm.at[pl.ds(base, _SC_CH)], xbuf1, sem1).wait()

        @pl.when(g < npair - 1)
        def _():
            pltpu.async_copy(
                x_hbm.at[pl.ds(base + (c0 + 2) * _SC_CH, _SC_CH)], xbuf0, sem0)

        compute_chunk(xbuf1, c0 + 1)
        return ()

    lax.fori_loop(0, npair, pair_body, ())


def _scred_kernel(consts_ref, p_ref, z8_ref, out_ref):
    alpha = consts_ref[0]
    p = p_ref[...]
    jj = jax.lax.broadcasted_iota(jnp.int32, (128, 128), 0)
    cc = jax.lax.broadcasted_iota(jnp.int32, (128, 128), 1)
    sel = jnp.where(jj // 16 == cc, 1.0, 0.0).astype(jnp.float32)
    z = jax.lax.dot_general(
        p, sel, (((1,), (0,)), ((), ())),
        preferred_element_type=jnp.float32)
    mu = jnp.exp(z)
    a = mu * mu
    num = a * _N_STUDY
    den = a * (_N_STUDY * _N_STUDY)
    e = (alpha * num) / den
    lane = jax.lax.broadcasted_iota(jnp.int32, z.shape, 1)
    esum = jnp.sum(jnp.where(lane < 8, e, 0.0))
    z8_ref[...] = z
    l2 = jax.lax.broadcasted_iota(jnp.int32, (1, 128), 1)
    out_ref[...] = jnp.where(l2 == 0, esum, 0.0)


def _nb_kernel(consts_ref, ibits_ref, z_ref, y_ref, out_ref):
    v = consts_ref[0]
    rmin_bits = ibits_ref[0]
    z = z_ref[...]
    mu = jnp.exp(z)
    a = mu * mu
    num = a * _N_STUDY
    den = a * (_N_STUDY * _N_STUDY)
    rr = (v * den) / num
    idx = jax.lax.bitcast_convert_type(rr, jnp.int32) - rmin_bits
    idx = jnp.clip(idx, 0, _NBINS - 1)

    p = num / (v * (mu * _N_STUDY) + num)
    yi = y_ref[...]
    yf = yi.astype(jnp.float32)
    nb = rr * jnp.log(1.0 - p) + yf * jnp.log(p)
    partial_nb = jnp.sum(nb)

    lane = jax.lax.broadcasted_iota(jnp.int32, (1, 128), 1)
    acc = jnp.full((1, 128), 0.0, jnp.float32)
    for j in range(_NBINS):
        mj = idx == j
        for k in (1, 2, 3):
            cnt = jnp.sum(jnp.where(mj & (yi == k), 1.0, 0.0))
            acc = jnp.where(lane == (j * 3 + k - 1), cnt, acc)
    acc = jnp.where(lane == _NBINS * 3, partial_nb, acc)
    out_ref[...] = acc.reshape(1, 1, 128)


def kernel(X, y, W, theta):
    n = jnp.float32(_N_STUDY)
    alpha = 100.0 * n * jax.nn.sigmoid(theta[0]) + 1e-8
    consts1 = jnp.stack([alpha, alpha]).astype(jnp.float32)

    nrows = X.shape[0]
    tc_rows = nrows - _SC_ROWS
    nblk = tc_rows // _BLK

    z3, part1 = pl.pallas_call(
        _matvec_kernel,
        grid=(nblk,),
        in_specs=[
            pl.BlockSpec(memory_space=pltpu.SMEM),
            pl.BlockSpec((1, 256), lambda i: (0, 0)),
            pl.BlockSpec((_BLK, 256), lambda i: (i, 0)),
        ],
        out_specs=[
            pl.BlockSpec((1, 1, _BLK), lambda i: (i, 0, 0)),
            pl.BlockSpec((1, 1, 128), lambda i: (i, 0, 0)),
        ],
        out_shape=[
            jax.ShapeDtypeStruct((nblk, 1, _BLK), jnp.float32),
            jax.ShapeDtypeStruct((nblk, 1, 128), jnp.float32),
        ],
        compiler_params=pltpu.CompilerParams(
            dimension_semantics=("parallel",)),
    )(consts1, W, X)

    sc_kernel = functools.partial(
        pl.kernel,
        out_type=jax.ShapeDtypeStruct((_SC_ROWS, 16), jnp.float32),
        mesh=plsc.VectorSubcoreMesh(core_axis_name="c", subcore_axis_name="s"),
        scratch_types=[
            pltpu.VMEM((_SC_CH, 256), jnp.float32),
            pltpu.VMEM((_SC_CH, 256), jnp.float32),
            pltpu.VMEM((_SC_CH, 16), jnp.float32),
            pltpu.VMEM((256,), jnp.float32),
            pltpu.SemaphoreType.DMA,
            pltpu.SemaphoreType.DMA,
        ],
    )(_sc_matvec)
    z_part = sc_kernel(X, W.reshape(256))

    z8, part_sc = pl.pallas_call(
        _scred_kernel,
        grid=(1,),
        in_specs=[
            pl.BlockSpec(memory_space=pltpu.SMEM),
            pl.BlockSpec((_SC_ROWS // 8, 128), lambda i: (0, 0)),
        ],
        out_specs=[
            pl.BlockSpec((_SC_ROWS // 8, 128), lambda i: (0, 0)),
            pl.BlockSpec((1, 128), lambda i: (0, 0)),
        ],
        out_shape=[
            jax.ShapeDtypeStruct((_SC_ROWS // 8, 128), jnp.float32),
            jax.ShapeDtypeStruct((1, 128), jnp.float32),
        ],
    )(consts1, z_part.reshape(_SC_ROWS // 8, 128))
    z_sc = z8[:, :8].reshape(_SC_ROWS)

    est_alpha = (jnp.sum(part1[:, 0, 0]) + part_sc[0, 0]) / jnp.float32(nrows)
    v = 1.0 / est_alpha
    r = v * n
    rc_bits = jax.lax.bitcast_convert_type(r, jnp.int32)
    rmin_bits = rc_bits - (_NBINS // 2)
    cand = jax.lax.bitcast_convert_type(
        rmin_bits + jnp.arange(_NBINS, dtype=jnp.int32), jnp.float32)
    ks = jnp.arange(1, 4, dtype=jnp.float32)
    T = (gammaln(cand[:, None] + ks[None, :])
         - gammaln(ks + 1.0)[None, :] - gammaln(cand)[:, None])

    consts2 = jnp.stack([v, r]).astype(jnp.float32)
    ibits = rmin_bits.reshape(1).astype(jnp.int32)

    nblk2 = nrows // _BLK2
    sub2 = _BLK2 // 128
    z_flat = jnp.concatenate([z3.reshape(tc_rows), z_sc])
    z2 = z_flat.reshape(nrows // 128, 128)
    y2 = y.reshape(nrows // 128, 128)

    partials = pl.pallas_call(
        _nb_kernel,
        grid=(nblk2,),
        in_specs=[
            pl.BlockSpec(memory_space=pltpu.SMEM),
            pl.BlockSpec(memory_space=pltpu.SMEM),
            pl.BlockSpec((sub2, 128), lambda i: (i, 0)),
            pl.BlockSpec((sub2, 128), lambda i: (i, 0)),
        ],
        out_specs=pl.BlockSpec((1, 1, 128), lambda i: (i, 0, 0)),
        out_shape=jax.ShapeDtypeStruct((nblk2, 1, 128), jnp.float32),
        compiler_params=pltpu.CompilerParams(
            dimension_semantics=("parallel",)),
    )(consts2, ibits, z2, y2)

    lanes = jnp.sum(partials[:, 0, :], axis=0)
    counts = lanes[: _NBINS * 3].reshape(_NBINS, 3)
    s3 = jnp.sum(counts * T)
    l = s3 + lanes[_NBINS * 3]
    return -l
